# baseline (speedup 1.0000x reference)
import jax
import jax.numpy as jnp
from jax.experimental import pallas as pl

B, N, C, H = 16, 577, 384, 6
NUM_PROP = 64
ALPHA = 0.1
HD = C // H
SCALE = HD ** (-0.5)
NUM_KEPT = N - 1 - NUM_PROP
NOUT = 1 + NUM_KEPT
NP = 640
K_ORDER = int(N * N * 0.75) + 1
ONE_BITS = 0x3F800000


def _layer_norm(x, w, b, eps=1e-5):
    mu = jnp.mean(x, axis=-1, keepdims=True)
    var = jnp.mean((x - mu) ** 2, axis=-1, keepdims=True)
    return (x - mu) / jnp.sqrt(var + eps) * w + b


def _dot(a, b):
    return jax.lax.dot_general(a, b, (((1,), (0,)), ((), ())),
                               preferred_element_type=jnp.float32)


def _dot_t(a, b):
    return jax.lax.dot_general(a, b, (((1,), (1,)), ((), ())),
                               preferred_element_type=jnp.float32)


def _stage1_kernel(x_ref, n1w_ref, n1b_ref, qkvw_ref, e_ref, v_ref):
    xb = x_ref[0]
    colm = jax.lax.broadcasted_iota(jnp.int32, (NP, NP), 1) < N
    h = _layer_norm(xb, n1w_ref[0], n1b_ref[0])
    qkv = _dot(h, qkvw_ref[...])
    v_ref[0] = qkv[:, 2 * C:3 * C]
    for hh in range(H):
        q = qkv[:, hh * HD:(hh + 1) * HD] * SCALE
        k = qkv[:, C + hh * HD:C + (hh + 1) * HD]
        logits = _dot_t(q, k)
        logits = jnp.where(colm, logits, -1e30)
        m = jnp.max(logits, axis=1, keepdims=True)
        e = jnp.exp(logits - m)
        e_ref[0, hh] = e[:N, :N]


def _stage2_kernel(e_ref, s_ref, x_ref, v_ref, projw_ref, projb_ref,
                   n2w_ref, n2b_ref, fc1w_ref, fc1b_ref, fc2w_ref, fc2b_ref,
                   out_ref):
    col2 = jax.lax.broadcasted_iota(jnp.int32, (N, N), 1)
    row2 = jax.lax.broadcasted_iota(jnp.int32, (N, N), 0)
    ident = (row2 == col2).astype(jnp.float32)

    def tocol(vrow):
        return jax.lax.dot_general(ident, vrow, (((1,), (1,)), ((), ())),
                                   preferred_element_type=jnp.float32)

    heads = []
    for hh in range(H):
        s_col = tocol(s_ref[0, hh:hh + 1, :])
        heads.append(e_ref[0, hh] / s_col)
    attn = jnp.stack(heads)

    bits = jax.lax.bitcast_convert_type(attn, jnp.int32)
    ihi = (bits >> 16).astype(jnp.int16)

    def count16(arr, thr16):
        mask = jnp.where(arr >= thr16, jnp.int16(1), jnp.int16(0))
        acc = mask[:, 0:72, :]
        for j in range(1, 8):
            acc = acc + mask[:, 72 * j:72 * (j + 1), :]
        acc = acc + jnp.pad(mask[:, 576:577, :], ((0, 0), (0, 71), (0, 0)))
        acc2 = acc[:, 0:8, :]
        for j in range(1, 9):
            acc2 = acc2 + acc[:, 8 * j:8 * (j + 1), :]
        return jnp.sum(acc2.astype(jnp.int32), axis=(1, 2)).reshape(H, 1, 1)

    lo = jnp.zeros((H, 1, 1), jnp.int32)
    hi = jnp.full((H, 1, 1), ONE_BITS >> 16, jnp.int32)

    def body16(_, lh):
        lo, hi = lh
        mid = lo + (hi - lo + 1) // 2
        cnt = count16(ihi, mid.astype(jnp.int16))
        pred = cnt >= K_ORDER
        return jnp.where(pred, mid, lo), jnp.where(pred, hi, mid - 1)

    g, _ = jax.lax.fori_loop(0, 14, body16, (lo, hi))
    base = count16(ihi, (g + 1).astype(jnp.int16))
    ilo = jnp.where(ihi == g.astype(jnp.int16),
                    ((bits & 0xFFFF) - 32768).astype(jnp.int16),
                    jnp.int16(-32768))
    lo = g << 16
    hi = lo | 0xFFFF

    def body32(_, lh):
        lo, hi = lh
        mid = lo + (hi - lo + 1) // 2
        t16 = ((mid & 0xFFFF) - 32768).astype(jnp.int16)
        cnt = base + count16(ilo, t16)
        pred = cnt >= K_ORDER
        return jnp.where(pred, mid, lo), jnp.where(pred, hi, mid - 1)

    lo, hi = jax.lax.fori_loop(0, 16, body32, (lo, hi))
    sigma = jax.lax.bitcast_convert_type(lo, jnp.float32)
    attn = jnp.where(attn >= sigma, attn, 0.0)

    vv = v_ref[0][:N]
    ctx = []
    for hh in range(H):
        ctx.append(_dot(attn[hh], vv[:, hh * HD:(hh + 1) * HD]))
    ctx = jnp.concatenate(ctx, axis=1)
    x1 = x_ref[0][:N] + _dot(ctx, projw_ref[...]) + projb_ref[0]

    c = attn[:, 0:1, :]
    val_row = (((((c[0] + c[1]) + c[2]) + c[3]) + c[4]) + c[5]) / 6.0
    val_col = tocol(val_row)
    before = ((val_col > val_row) |
              ((val_col == val_row) & (row2 < col2))) & (row2 >= 1)
    rank = jnp.sum(before.astype(jnp.int32), axis=0, keepdims=True)

    rank_b = jnp.broadcast_to(rank, (N, N))
    pk2 = jnp.where(((row2 == 0) & (col2 == 0)) |
                    ((row2 >= 1) & (row2 <= NUM_KEPT) & (col2 >= 1) &
                     (rank_b == row2 - 1)), 1.0, 0.0)
    elim = ((col2[0:1, :] >= 1) & (rank >= NUM_KEPT)).astype(jnp.float32)

    xpf = []
    for hh in range(H):
        am = attn[hh] * elim
        xpf.append(_dot(am, x1[:, hh * HD:(hh + 1) * HD]))
    xpf = jnp.concatenate(xpf, axis=1)
    rmask = (jax.lax.broadcasted_iota(jnp.int32, (N, 1), 0) >= 1)
    xpf = xpf * rmask.astype(jnp.float32)
    x2 = _dot(pk2, x1 + ALPHA * xpf)

    h2 = _layer_norm(x2, n2w_ref[0], n2b_ref[0])
    a1 = _dot(h2.astype(jnp.bfloat16), fc1w_ref[...]) + fc1b_ref[0]
    g = 0.5 * a1 * (1.0 + jax.lax.erf(a1 * (2.0 ** -0.5)))
    o2 = _dot(g.astype(jnp.bfloat16), fc2w_ref[...]) + fc2b_ref[0]
    x3 = x2 + o2
    out_ref[0] = x3[:NOUT]


@jax.jit
def kernel(x, norm1_w, norm1_b, qkv_w, proj_w, proj_b, norm2_w, norm2_b,
           fc1_w, fc1_b, fc2_w, fc2_b):
    xp = jnp.pad(x, ((0, 0), (0, NP - N), (0, 0)))

    def vec(a):
        return a.reshape(1, -1)

    def full(shape):
        return pl.BlockSpec(shape, lambda b: (0,) * len(shape))

    e, v = pl.pallas_call(
        _stage1_kernel,
        grid=(B,),
        in_specs=[
            pl.BlockSpec((1, NP, C), lambda b: (b, 0, 0)),
            full((1, C)), full((1, C)), full((C, 3 * C)),
        ],
        out_specs=[
            pl.BlockSpec((1, H, N, N), lambda b: (b, 0, 0, 0)),
            pl.BlockSpec((1, NP, C), lambda b: (b, 0, 0)),
        ],
        out_shape=[
            jax.ShapeDtypeStruct((B, H, N, N), jnp.float32),
            jax.ShapeDtypeStruct((B, NP, C), jnp.float32),
        ],
    )(xp, vec(norm1_w), vec(norm1_b), qkv_w.T)

    s = jnp.sum(e, axis=-1)

    out = pl.pallas_call(
        _stage2_kernel,
        grid=(B,),
        in_specs=[
            pl.BlockSpec((1, H, N, N), lambda b: (b, 0, 0, 0)),
            pl.BlockSpec((1, H, N), lambda b: (b, 0, 0)),
            pl.BlockSpec((1, NP, C), lambda b: (b, 0, 0)),
            pl.BlockSpec((1, NP, C), lambda b: (b, 0, 0)),
            full((C, C)), full((1, C)),
            full((1, C)), full((1, C)),
            full((C, 4 * C)), full((1, 4 * C)),
            full((4 * C, C)), full((1, C)),
        ],
        out_specs=pl.BlockSpec((1, NOUT, C), lambda b: (b, 0, 0)),
        out_shape=jax.ShapeDtypeStruct((B, NOUT, C), jnp.float32),
    )(e, s, xp, v, proj_w.T, vec(proj_b), vec(norm2_w), vec(norm2_b),
      fc1_w.T.astype(jnp.bfloat16), vec(fc1_b),
      fc2_w.T.astype(jnp.bfloat16), vec(fc2_b))
    return out


# speedup vs baseline: 1.0029x; 1.0029x over previous
import jax
import jax.numpy as jnp
from jax.experimental import pallas as pl

B, N, C, H = 16, 577, 384, 6
NUM_PROP = 64
ALPHA = 0.1
HD = C // H
SCALE = HD ** (-0.5)
NUM_KEPT = N - 1 - NUM_PROP
NOUT = 1 + NUM_KEPT
NP = 640
K_ORDER = int(N * N * 0.75) + 1
ONE_BITS = 0x3F800000


def _layer_norm(x, w, b, eps=1e-5):
    mu = jnp.mean(x, axis=-1, keepdims=True)
    var = jnp.mean((x - mu) ** 2, axis=-1, keepdims=True)
    return (x - mu) / jnp.sqrt(var + eps) * w + b


def _dot(a, b):
    return jax.lax.dot_general(a, b, (((1,), (0,)), ((), ())),
                               preferred_element_type=jnp.float32)


def _dot_t(a, b):
    return jax.lax.dot_general(a, b, (((1,), (1,)), ((), ())),
                               preferred_element_type=jnp.float32)


def _stage1_kernel(x_ref, n1w_ref, n1b_ref, qkvw_ref, e_ref, v_ref):
    xb = x_ref[0]
    colm = jax.lax.broadcasted_iota(jnp.int32, (NP, NP), 1) < N
    h = _layer_norm(xb, n1w_ref[0], n1b_ref[0])
    qkv = _dot(h, qkvw_ref[...])
    v_ref[0] = qkv[:, 2 * C:3 * C]
    for hh in range(H):
        q = qkv[:, hh * HD:(hh + 1) * HD] * SCALE
        k = qkv[:, C + hh * HD:C + (hh + 1) * HD]
        logits = _dot_t(q, k)
        logits = jnp.where(colm, logits, -1e30)
        m = jnp.max(logits, axis=1, keepdims=True)
        e = jnp.exp(logits - m)
        e_ref[0, hh] = e[:N, :N]


def _stage2_kernel(e_ref, s_ref, x_ref, v_ref, projw_ref, projb_ref,
                   n2w_ref, n2b_ref, fc1w_ref, fc1b_ref, fc2w_ref, fc2b_ref,
                   out_ref):
    col2 = jax.lax.broadcasted_iota(jnp.int32, (N, N), 1)
    row2 = jax.lax.broadcasted_iota(jnp.int32, (N, N), 0)
    ident = (row2 == col2).astype(jnp.float32)

    def tocol(vrow):
        return jax.lax.dot_general(ident, vrow, (((1,), (1,)), ((), ())),
                                   preferred_element_type=jnp.float32)

    heads = []
    for hh in range(H):
        s_col = tocol(s_ref[0, hh:hh + 1, :])
        heads.append(e_ref[0, hh] / s_col)
    attn = jnp.stack(heads)

    bits = jax.lax.bitcast_convert_type(attn, jnp.int32)
    ihi = (bits >> 16).astype(jnp.int16)

    def count16(arr, thr16):
        mask = jnp.where(arr >= thr16, jnp.int16(1), jnp.int16(0))
        acc = mask[:, 0:72, :]
        for j in range(1, 8):
            acc = acc + mask[:, 72 * j:72 * (j + 1), :]
        acc = acc + jnp.pad(mask[:, 576:577, :], ((0, 0), (0, 71), (0, 0)))
        acc2 = acc[:, 0:8, :]
        for j in range(1, 9):
            acc2 = acc2 + acc[:, 8 * j:8 * (j + 1), :]
        return jnp.sum(acc2.astype(jnp.int32), axis=(1, 2)).reshape(H, 1, 1)

    lo = jnp.zeros((H, 1, 1), jnp.int32)
    hi = jnp.full((H, 1, 1), ONE_BITS >> 16, jnp.int32)

    def body16(_, lh):
        lo, hi = lh
        mid = lo + (hi - lo + 1) // 2
        cnt = count16(ihi, mid.astype(jnp.int16))
        pred = cnt >= K_ORDER
        return jnp.where(pred, mid, lo), jnp.where(pred, hi, mid - 1)

    g, _ = jax.lax.fori_loop(0, 14, body16, (lo, hi))
    base = count16(ihi, (g + 1).astype(jnp.int16))
    ilo = jnp.where(ihi == g.astype(jnp.int16),
                    ((bits & 0xFFFF) - 32768).astype(jnp.int16),
                    jnp.int16(-32768))
    lo = g << 16
    hi = lo | 0xFFFF

    def body32(_, lh):
        lo, hi = lh
        mid = lo + (hi - lo + 1) // 2
        t16 = ((mid & 0xFFFF) - 32768).astype(jnp.int16)
        cnt = base + count16(ilo, t16)
        pred = cnt >= K_ORDER
        return jnp.where(pred, mid, lo), jnp.where(pred, hi, mid - 1)

    lo, hi = jax.lax.fori_loop(0, 16, body32, (lo, hi))
    sigma = jax.lax.bitcast_convert_type(lo, jnp.float32)
    attn = jnp.where(attn >= sigma, attn, 0.0)

    vv = v_ref[0][:N]
    ctx = []
    for hh in range(H):
        ctx.append(_dot(attn[hh], vv[:, hh * HD:(hh + 1) * HD]))
    ctx = jnp.concatenate(ctx, axis=1)
    x1 = x_ref[0][:N] + _dot(ctx, projw_ref[...]) + projb_ref[0]

    c = attn[:, 0:1, :]
    val_row = (((((c[0] + c[1]) + c[2]) + c[3]) + c[4]) + c[5]) / 6.0
    val_col = tocol(val_row)
    before = ((val_col > val_row) |
              ((val_col == val_row) & (row2 < col2))) & (row2 >= 1)
    rank = jnp.sum(before.astype(jnp.int32), axis=0, keepdims=True)

    rank_b = jnp.broadcast_to(rank, (N, N))
    pk2 = jnp.where(((row2 == 0) & (col2 == 0)) |
                    ((row2 >= 1) & (row2 <= NUM_KEPT) & (col2 >= 1) &
                     (rank_b == row2 - 1)), 1.0, 0.0)
    elim = ((col2[0:1, :] >= 1) & (rank >= NUM_KEPT)).astype(jnp.float32)

    xpf = []
    for hh in range(H):
        am = attn[hh] * elim
        xpf.append(_dot(am, x1[:, hh * HD:(hh + 1) * HD]))
    xpf = jnp.concatenate(xpf, axis=1)
    rmask = (jax.lax.broadcasted_iota(jnp.int32, (N, 1), 0) >= 1)
    xpf = xpf * rmask.astype(jnp.float32)
    x2 = _dot(pk2, x1 + ALPHA * xpf)

    h2 = _layer_norm(x2, n2w_ref[0], n2b_ref[0])
    a1 = _dot(h2, fc1w_ref[...]) + fc1b_ref[0]
    g = 0.5 * a1 * (1.0 + jax.lax.erf(a1 * (2.0 ** -0.5)))
    o2 = _dot(g, fc2w_ref[...]) + fc2b_ref[0]
    x3 = x2 + o2
    out_ref[0] = x3[:NOUT]


@jax.jit
def kernel(x, norm1_w, norm1_b, qkv_w, proj_w, proj_b, norm2_w, norm2_b,
           fc1_w, fc1_b, fc2_w, fc2_b):
    xp = jnp.pad(x, ((0, 0), (0, NP - N), (0, 0)))

    def vec(a):
        return a.reshape(1, -1)

    def full(shape):
        return pl.BlockSpec(shape, lambda b: (0,) * len(shape))

    e, v = pl.pallas_call(
        _stage1_kernel,
        grid=(B,),
        in_specs=[
            pl.BlockSpec((1, NP, C), lambda b: (b, 0, 0)),
            full((1, C)), full((1, C)), full((C, 3 * C)),
        ],
        out_specs=[
            pl.BlockSpec((1, H, N, N), lambda b: (b, 0, 0, 0)),
            pl.BlockSpec((1, NP, C), lambda b: (b, 0, 0)),
        ],
        out_shape=[
            jax.ShapeDtypeStruct((B, H, N, N), jnp.float32),
            jax.ShapeDtypeStruct((B, NP, C), jnp.float32),
        ],
    )(xp, vec(norm1_w), vec(norm1_b), qkv_w.T)

    s = jnp.sum(e, axis=-1)

    out = pl.pallas_call(
        _stage2_kernel,
        grid=(B,),
        in_specs=[
            pl.BlockSpec((1, H, N, N), lambda b: (b, 0, 0, 0)),
            pl.BlockSpec((1, H, N), lambda b: (b, 0, 0)),
            pl.BlockSpec((1, NP, C), lambda b: (b, 0, 0)),
            pl.BlockSpec((1, NP, C), lambda b: (b, 0, 0)),
            full((C, C)), full((1, C)),
            full((1, C)), full((1, C)),
            full((C, 4 * C)), full((1, 4 * C)),
            full((4 * C, C)), full((1, C)),
        ],
        out_specs=pl.BlockSpec((1, NOUT, C), lambda b: (b, 0, 0)),
        out_shape=jax.ShapeDtypeStruct((B, NOUT, C), jnp.float32),
    )(e, s, xp, v, proj_w.T, vec(proj_b), vec(norm2_w), vec(norm2_b),
      fc1_w.T, vec(fc1_b), fc2_w.T, vec(fc2_b))
    return out
